# baseline (speedup 1.0000x reference)
import functools

import jax
import jax.numpy as jnp
from jax import lax
from jax.experimental import pallas as pl
from jax.experimental.pallas import tpu as pltpu
from jax.experimental.pallas import tpu_sc as plsc

B = 16384
DIM = 10
YEAR_BINS = 20
H1 = 64
H2 = 32

NC = 2
NS = 16
NW = NC * NS
PER_W = B // NW


def _title_gather(idx, table):
    mesh = plsc.VectorSubcoreMesh(core_axis_name="c", subcore_axis_name="s")

    @functools.partial(
        pl.kernel,
        mesh=mesh,
        out_type=jax.ShapeDtypeStruct((B, DIM), jnp.float32),
        scratch_types=[
            pltpu.VMEM((PER_W,), jnp.int32),
            pltpu.VMEM((PER_W, DIM), jnp.float32),
            pltpu.SemaphoreType.DMA,
        ],
    )
    def k(idx_hbm, table_hbm, out_hbm, idx_v, rows_v, sem):
        wid = lax.axis_index("s") * NC + lax.axis_index("c")
        base = wid * PER_W
        pltpu.sync_copy(idx_hbm.at[pl.ds(base, PER_W)], idx_v)

        def fire(g, carry):
            chunk = idx_v[pl.ds(g * 16, 16)]
            for j in range(16):
                pltpu.async_copy(
                    table_hbm.at[pl.ds(chunk[j], 1)],
                    rows_v.at[pl.ds(g * 16 + j, 1)], sem)
            return carry

        lax.fori_loop(0, PER_W // 16, fire, 0)

        def drain(i, carry):
            pltpu.make_async_copy(
                table_hbm.at[pl.ds(0, 1)], rows_v.at[pl.ds(i, 1)], sem).wait()
            return carry

        lax.fori_loop(0, PER_W, drain, 0)
        pltpu.sync_copy(rows_v, out_hbm.at[pl.ds(base, PER_W)])

    return k(idx, table)


def _mlp_body(te_ref, yi_ref, ni_ref, yt_ref, nt_ref,
              w1a_ref, w1b_ref, w1c_ref, b1_ref, w2_ref, b2_ref, out_ref):
    iota = lax.broadcasted_iota(jnp.int32, (1, YEAR_BINS), 1)
    oh_y = (yi_ref[...] == iota).astype(jnp.float32)
    oh_n = (ni_ref[...] == iota).astype(jnp.float32)
    ye = jnp.dot(oh_y, yt_ref[...], preferred_element_type=jnp.float32)
    ne = jnp.dot(oh_n, nt_ref[...], preferred_element_type=jnp.float32)
    h = jnp.dot(te_ref[...], w1a_ref[...], preferred_element_type=jnp.float32)
    h = h + jnp.dot(ye, w1b_ref[...], preferred_element_type=jnp.float32)
    h = h + jnp.dot(ne, w1c_ref[...], preferred_element_type=jnp.float32)
    h = jnp.maximum(h + b1_ref[...], 0.0)
    out_ref[...] = (
        jnp.dot(h, w2_ref[...], preferred_element_type=jnp.float32) + b2_ref[...]
    )


def _mlp(te, yi2, ni2, yt, nt, w1a, w1b, w1c, b1r, w2, b2r):
    BM = 2048
    grid = (B // BM,)
    return pl.pallas_call(
        _mlp_body,
        grid=grid,
        in_specs=[
            pl.BlockSpec((BM, DIM), lambda i: (i, 0)),
            pl.BlockSpec((BM, 1), lambda i: (i, 0)),
            pl.BlockSpec((BM, 1), lambda i: (i, 0)),
            pl.BlockSpec((YEAR_BINS, DIM), lambda i: (0, 0)),
            pl.BlockSpec((YEAR_BINS, DIM), lambda i: (0, 0)),
            pl.BlockSpec((DIM, H1), lambda i: (0, 0)),
            pl.BlockSpec((DIM, H1), lambda i: (0, 0)),
            pl.BlockSpec((DIM, H1), lambda i: (0, 0)),
            pl.BlockSpec((1, H1), lambda i: (0, 0)),
            pl.BlockSpec((H1, H2), lambda i: (0, 0)),
            pl.BlockSpec((1, H2), lambda i: (0, 0)),
        ],
        out_specs=pl.BlockSpec((BM, H2), lambda i: (i, 0)),
        out_shape=jax.ShapeDtypeStruct((B, H2), jnp.float32),
    )(te, yi2, ni2, yt, nt, w1a, w1b, w1c, b1r, w2, b2r)


def kernel(title_idx, year_idx, num_ratings_idx, title_table, year_table,
           nr_table, W1, b1, W2, b2):
    te = _title_gather(title_idx.astype(jnp.int32), title_table)
    yi2 = year_idx.astype(jnp.int32).reshape(B, 1)
    ni2 = num_ratings_idx.astype(jnp.int32).reshape(B, 1)
    return _mlp(
        te, yi2, ni2, year_table, nr_table,
        W1[0:DIM], W1[DIM:2 * DIM], W1[2 * DIM:3 * DIM],
        b1.reshape(1, H1), W2, b2.reshape(1, H2),
    )


# speedup vs baseline: 1.0054x; 1.0054x over previous
import functools

import jax
import jax.numpy as jnp
from jax import lax
from jax.experimental import pallas as pl
from jax.experimental.pallas import tpu as pltpu
from jax.experimental.pallas import tpu_sc as plsc

B = 16384
DIM = 10
YEAR_BINS = 20
H1 = 64
H2 = 32

NC = 2
NS = 16
NW = NC * NS
PER_W = B // NW


def _title_gather(idx, table):
    mesh = plsc.VectorSubcoreMesh(core_axis_name="c", subcore_axis_name="s")

    @functools.partial(
        pl.kernel,
        mesh=mesh,
        out_type=jax.ShapeDtypeStruct((B, DIM), jnp.float32),
        scratch_types=[
            pltpu.VMEM((PER_W,), jnp.int32),
            pltpu.VMEM((PER_W, DIM), jnp.float32),
            pltpu.SemaphoreType.DMA,
        ],
    )
    def k(idx_hbm, table_hbm, out_hbm, idx_v, rows_v, sem):
        wid = lax.axis_index("s") * NC + lax.axis_index("c")
        base = wid * PER_W
        pltpu.sync_copy(idx_hbm.at[pl.ds(base, PER_W)], idx_v)

        def fire(i, carry):
            r = idx_v[pl.ds(i, 1)][0]
            pltpu.async_copy(
                table_hbm.at[pl.ds(r, 1)], rows_v.at[pl.ds(i, 1)], sem)
            return carry

        lax.fori_loop(0, PER_W, fire, 0)

        def drain(i, carry):
            pltpu.make_async_copy(
                table_hbm.at[pl.ds(0, 1)], rows_v.at[pl.ds(i, 1)], sem).wait()
            return carry

        lax.fori_loop(0, PER_W, drain, 0)
        pltpu.sync_copy(rows_v, out_hbm.at[pl.ds(base, PER_W)])

    return k(idx, table)


def _mlp_body(te_ref, yi_ref, ni_ref, yt_ref, nt_ref,
              w1a_ref, w1b_ref, w1c_ref, b1_ref, w2_ref, b2_ref, out_ref):
    iota = lax.broadcasted_iota(jnp.int32, (1, YEAR_BINS), 1)
    oh_y = (yi_ref[...] == iota).astype(jnp.float32)
    oh_n = (ni_ref[...] == iota).astype(jnp.float32)
    ye = jnp.dot(oh_y, yt_ref[...], preferred_element_type=jnp.float32)
    ne = jnp.dot(oh_n, nt_ref[...], preferred_element_type=jnp.float32)
    h = jnp.dot(te_ref[...], w1a_ref[...], preferred_element_type=jnp.float32)
    h = h + jnp.dot(ye, w1b_ref[...], preferred_element_type=jnp.float32)
    h = h + jnp.dot(ne, w1c_ref[...], preferred_element_type=jnp.float32)
    h = jnp.maximum(h + b1_ref[...], 0.0)
    out_ref[...] = (
        jnp.dot(h, w2_ref[...], preferred_element_type=jnp.float32) + b2_ref[...]
    )


def _mlp(te, yi2, ni2, yt, nt, w1a, w1b, w1c, b1r, w2, b2r):
    BM = 2048
    grid = (B // BM,)
    return pl.pallas_call(
        _mlp_body,
        grid=grid,
        in_specs=[
            pl.BlockSpec((BM, DIM), lambda i: (i, 0)),
            pl.BlockSpec((BM, 1), lambda i: (i, 0)),
            pl.BlockSpec((BM, 1), lambda i: (i, 0)),
            pl.BlockSpec((YEAR_BINS, DIM), lambda i: (0, 0)),
            pl.BlockSpec((YEAR_BINS, DIM), lambda i: (0, 0)),
            pl.BlockSpec((DIM, H1), lambda i: (0, 0)),
            pl.BlockSpec((DIM, H1), lambda i: (0, 0)),
            pl.BlockSpec((DIM, H1), lambda i: (0, 0)),
            pl.BlockSpec((1, H1), lambda i: (0, 0)),
            pl.BlockSpec((H1, H2), lambda i: (0, 0)),
            pl.BlockSpec((1, H2), lambda i: (0, 0)),
        ],
        out_specs=pl.BlockSpec((BM, H2), lambda i: (i, 0)),
        out_shape=jax.ShapeDtypeStruct((B, H2), jnp.float32),
    )(te, yi2, ni2, yt, nt, w1a, w1b, w1c, b1r, w2, b2r)


def kernel(title_idx, year_idx, num_ratings_idx, title_table, year_table,
           nr_table, W1, b1, W2, b2):
    te = _title_gather(title_idx.astype(jnp.int32), title_table)
    yi2 = year_idx.astype(jnp.int32).reshape(B, 1)
    ni2 = num_ratings_idx.astype(jnp.int32).reshape(B, 1)
    return _mlp(
        te, yi2, ni2, year_table, nr_table,
        W1[0:DIM], W1[DIM:2 * DIM], W1[2 * DIM:3 * DIM],
        b1.reshape(1, H1), W2, b2.reshape(1, H2),
    )
